# baseline (speedup 1.0000x reference)
import functools

import jax
import jax.numpy as jnp
from jax import lax
from jax.experimental import pallas as pl
from jax.experimental.pallas import tpu as pltpu


def _body(nk, nv, k_block, v_block, vocab, n_ctx, d,
          idx_ref, embT_ref, w1_ref, b1_ref, w2_ref, b2_ref, lp_ref,
          bufs, eflat, hid_s, m_s, s_s, *sems):
    i = pl.program_id(0)
    rows_per_k = n_ctx // nk

    def _copy(r):
        base = pl.multiple_of((idx_ref[r] // 128) * 128, 128)
        return pltpu.make_async_copy(
            embT_ref.at[:, pl.ds(base, 128)],
            bufs.at[:, pl.ds(r * 128, 128)], sems[r // rows_per_k])

    @pl.when(i == 0)
    def _issue():
        for r in range(n_ctx):
            _copy(r).start()

    for sk in range(nk):
        @pl.when(i == sk)
        def _w1(k=sk):
            @pl.when(i == 0)
            def _():
                hid_s[...] = b1_ref[...]

            for r in range(k * rows_per_k, (k + 1) * rows_per_k):
                _copy(r).wait()
            lane = lax.broadcasted_iota(jnp.int32, (1, 128), 1)
            for p in range(k * rows_per_k // 2, (k + 1) * rows_per_k // 2):
                segs = []
                for r in (2 * p, 2 * p + 1):
                    off = idx_ref[r] - (idx_ref[r] // 128) * 128
                    oh = (lane == off).astype(jnp.float32)
                    segs.append(lax.dot_general(
                        oh, bufs[:, pl.ds(r * 128, 128)],
                        (((1,), (1,)), ((), ())),
                        preferred_element_type=jnp.float32))
                eflat[0, pl.ds(p * 2 * d, 2 * d)] = (
                    jnp.concatenate(segs, axis=1)[0])

            hid_s[...] += lax.dot_general(
                eflat[:, pl.ds(k * k_block, k_block)], w1_ref[...],
                (((1,), (1,)), ((), ())),
                preferred_element_type=jnp.float32)

            @pl.when(i == nk - 1)
            def _():
                hid_s[...] = jnp.maximum(hid_s[...], 0.0)

    @pl.when((i >= nk) & (i < nk + nv))
    def _w2():
        j = i - nk
        o = lax.dot_general(
            hid_s[...], w2_ref[...], (((1,), (1,)), ((), ())),
            preferred_element_type=jnp.float32) + b2_ref[...]
        lp_ref[0, pl.ds(j * v_block, v_block)] = o[0]
        col = j * v_block + lax.broadcasted_iota(jnp.int32, o.shape, 1)
        om = jnp.where(col < vocab, o, -jnp.inf)
        t = jnp.max(om, keepdims=True)

        @pl.when(j == 0)
        def _():
            m_s[...] = t
            s_s[...] = jnp.sum(jnp.exp(om - t), keepdims=True)

        @pl.when(j > 0)
        def _():
            m_old = m_s[...]
            m_new = jnp.maximum(m_old, t)
            m_s[...] = m_new
            s_s[...] = (s_s[...] * jnp.exp(m_old - m_new)
                        + jnp.sum(jnp.exp(om - m_new), keepdims=True))

    @pl.when(i == nk + nv)
    def _finish():
        lp_ref[...] = lp_ref[...] - (m_s[...] + jnp.log(s_s[...]))


def kernel(inputs, emb, W1, b1, W2, b2):
    idx = inputs.astype(jnp.int32)
    embT = emb.T
    n_ctx = idx.shape[0]
    d = embT.shape[0]
    vocab, hidden = W2.shape
    in1 = W1.shape[1]
    k_block = 1280
    v_block = 8192
    nk = in1 // k_block
    nv = -(-vocab // v_block)
    v_pad = nv * v_block
    grid = nk + nv + 1

    def w1_idx(i):
        return (0, jnp.clip(i, 0, nk - 1))

    def w2_idx(i):
        return (jnp.clip(i - nk, 0, nv - 1), 0)

    def b2_idx(i):
        return (0, jnp.clip(i - nk, 0, nv - 1))

    lp = pl.pallas_call(
        functools.partial(_body, nk, nv, k_block, v_block, vocab, n_ctx, d),
        grid=(grid,),
        in_specs=[
            pl.BlockSpec(memory_space=pltpu.SMEM),
            pl.BlockSpec(memory_space=pl.ANY),
            pl.BlockSpec((hidden, k_block), w1_idx),
            pl.BlockSpec((1, hidden), lambda i: (0, 0)),
            pl.BlockSpec((v_block, hidden), w2_idx),
            pl.BlockSpec((1, v_block), b2_idx),
        ],
        out_specs=pl.BlockSpec((1, v_pad), lambda i: (0, 0)),
        out_shape=jax.ShapeDtypeStruct((1, v_pad), jnp.float32),
        scratch_shapes=[
            pltpu.VMEM((d, n_ctx * 128), jnp.float32),
            pltpu.VMEM((1, n_ctx * d), jnp.float32),
            pltpu.VMEM((1, hidden), jnp.float32),
            pltpu.VMEM((1, 1), jnp.float32),
            pltpu.VMEM((1, 1), jnp.float32),
        ] + [pltpu.SemaphoreType.DMA] * nk,
    )(idx, embT, W1, b1.reshape(1, -1), W2, b2.reshape(1, -1))
    return lp[:, :vocab]


# speedup vs baseline: 1.0228x; 1.0228x over previous
import functools

import jax
import jax.numpy as jnp
from jax import lax
from jax.experimental import pallas as pl
from jax.experimental.pallas import tpu as pltpu


def _body(nk, nv, k_block, v_block, vocab, n_ctx, d,
          idx_ref, embT_ref, w1_ref, b1_ref, w2_ref, b2_ref, lp_ref,
          bufs, eflat, hid_s, m_s, s_s, *sems):
    i = pl.program_id(0)
    rows_per_k = n_ctx // nk

    def _copy(r):
        base = pl.multiple_of((idx_ref[r] // 128) * 128, 128)
        return pltpu.make_async_copy(
            embT_ref.at[:, pl.ds(base, 128)],
            bufs.at[:, pl.ds(r * 128, 128)], sems[r // rows_per_k])

    @pl.when(i == 0)
    def _issue():
        for r in range(n_ctx):
            _copy(r).start()

    for sk in range(nk):
        @pl.when(i == sk)
        def _w1(k=sk):
            @pl.when(i == 0)
            def _():
                hid_s[...] = b1_ref[...]

            for r in range(k * rows_per_k, (k + 1) * rows_per_k):
                _copy(r).wait()
            lane = lax.broadcasted_iota(jnp.int32, (1, 128), 1)
            for p in range(k * rows_per_k // 2, (k + 1) * rows_per_k // 2):
                segs = []
                for r in (2 * p, 2 * p + 1):
                    off = idx_ref[r] - (idx_ref[r] // 128) * 128
                    oh = (lane == off).astype(jnp.float32)
                    segs.append(lax.dot_general(
                        oh, bufs[:, pl.ds(r * 128, 128)],
                        (((1,), (1,)), ((), ())),
                        preferred_element_type=jnp.float32))
                eflat[0, pl.ds(p * 2 * d, 2 * d)] = (
                    jnp.concatenate(segs, axis=1)[0])

            hid_s[...] += lax.dot_general(
                eflat[:, pl.ds(k * k_block, k_block)], w1_ref[...],
                (((1,), (1,)), ((), ())),
                preferred_element_type=jnp.float32)

            @pl.when(i == nk - 1)
            def _():
                hid_s[...] = jnp.maximum(hid_s[...], 0.0)

    @pl.when((i >= nk) & (i < nk + nv))
    def _w2():
        j = i - nk
        o = lax.dot_general(
            hid_s[...], w2_ref[...], (((1,), (1,)), ((), ())),
            preferred_element_type=jnp.float32) + b2_ref[...]
        lp_ref[0, pl.ds(j * v_block, v_block)] = o[0]
        col = j * v_block + lax.broadcasted_iota(jnp.int32, o.shape, 1)
        om = jnp.where(col < vocab, o, -jnp.inf)
        t = jnp.max(om, keepdims=True)

        @pl.when(j == 0)
        def _():
            m_s[...] = t
            s_s[...] = jnp.sum(jnp.exp(om - t), keepdims=True)

        @pl.when(j > 0)
        def _():
            m_old = m_s[...]
            m_new = jnp.maximum(m_old, t)
            m_s[...] = m_new
            s_s[...] = (s_s[...] * jnp.exp(m_old - m_new)
                        + jnp.sum(jnp.exp(om - m_new), keepdims=True))

    @pl.when(i == nk + nv)
    def _finish():
        lp_ref[...] = lp_ref[...] - (m_s[...] + jnp.log(s_s[...]))


def kernel(inputs, emb, W1, b1, W2, b2):
    idx = inputs.astype(jnp.int32)
    embT = emb.T
    n_ctx = idx.shape[0]
    d = embT.shape[0]
    vocab, hidden = W2.shape
    in1 = W1.shape[1]
    k_block = 2560
    v_block = 8192
    nk = in1 // k_block
    nv = -(-vocab // v_block)
    v_pad = nv * v_block
    grid = nk + nv + 1

    def w1_idx(i):
        return (0, jnp.clip(i, 0, nk - 1))

    def w2_idx(i):
        return (jnp.clip(i - nk, 0, nv - 1), 0)

    def b2_idx(i):
        return (0, jnp.clip(i - nk, 0, nv - 1))

    lp = pl.pallas_call(
        functools.partial(_body, nk, nv, k_block, v_block, vocab, n_ctx, d),
        grid=(grid,),
        in_specs=[
            pl.BlockSpec(memory_space=pltpu.SMEM),
            pl.BlockSpec(memory_space=pl.ANY),
            pl.BlockSpec((hidden, k_block), w1_idx),
            pl.BlockSpec((1, hidden), lambda i: (0, 0)),
            pl.BlockSpec((v_block, hidden), w2_idx),
            pl.BlockSpec((1, v_block), b2_idx),
        ],
        out_specs=pl.BlockSpec((1, v_pad), lambda i: (0, 0)),
        out_shape=jax.ShapeDtypeStruct((1, v_pad), jnp.float32),
        scratch_shapes=[
            pltpu.VMEM((d, n_ctx * 128), jnp.float32),
            pltpu.VMEM((1, n_ctx * d), jnp.float32),
            pltpu.VMEM((1, hidden), jnp.float32),
            pltpu.VMEM((1, 1), jnp.float32),
            pltpu.VMEM((1, 1), jnp.float32),
        ] + [pltpu.SemaphoreType.DMA] * nk,
    )(idx, embT, W1, b1.reshape(1, -1), W2, b2.reshape(1, -1))
    return lp[:, :vocab]
